# baseline (speedup 1.0000x reference)
import dataclasses
import functools

import jax
import jax.numpy as jnp
from jax import lax
from jax.experimental import pallas as pl
from jax.experimental.pallas import tpu as pltpu
from jax.experimental.pallas import tpu_sc as plsc

EMBED = 64
N_CTX = 50
N_NEG = 20
N_OUT = N_NEG + 1
LANES = 16
CHUNKS = EMBED // LANES


def _softplus16(t):
    m = jnp.maximum(t, 0.0)
    u = jnp.exp(-jnp.abs(t))
    y = 1.0 + u
    x = u * (6.0 + u) / (6.0 + 4.0 * u)
    for _ in range(3):
        x = x + y * jnp.exp(-x) - 1.0
    return m + x


def _sc_body(ctx_idx_hbm, wout_idx_hbm, w_in_hbm, w_out_hbm, out_hbm,
             ctx_idx_v, wout_idx_v, ctx_rows_v, wout_rows_v, out_v,
             sem_a, sem_b):
    wid = lax.axis_index("s") * 2 + lax.axis_index("c")

    @pl.when(wid == 0)
    def _():
        cp_a = pltpu.async_copy(ctx_idx_hbm, ctx_idx_v, sem_a)
        cp_b = pltpu.async_copy(wout_idx_hbm, wout_idx_v, sem_b)
        cp_a.wait()
        cp_b.wait()
        g_a = pltpu.async_copy(w_in_hbm.at[ctx_idx_v], ctx_rows_v, sem_a)
        g_b = pltpu.async_copy(w_out_hbm.at[wout_idx_v], wout_rows_v, sem_b)
        g_a.wait()
        g_b.wait()

        h = []
        for c in range(CHUNKS):
            acc = ctx_rows_v[0, pl.ds(LANES * c, LANES)]
            for i in range(1, N_CTX):
                acc = acc + ctx_rows_v[i, pl.ds(LANES * c, LANES)]
            h.append(acc * (1.0 / N_CTX))

        svals = []
        for j in range(N_OUT):
            p = wout_rows_v[j, pl.ds(0, LANES)] * h[0]
            for c in range(1, CHUNKS):
                p = p + wout_rows_v[j, pl.ds(LANES * c, LANES)] * h[c]
            svals.append(jnp.sum(p))

        io = lax.iota(jnp.int32, LANES)
        t0 = jnp.full((LANES,), -100.0, jnp.float32)
        t1 = jnp.full((LANES,), -100.0, jnp.float32)
        t0 = jnp.where(io == 0, -svals[0], t0)
        for j in range(1, LANES):
            t0 = jnp.where(io == j, svals[j], t0)
        for j in range(LANES, N_OUT):
            t1 = jnp.where(io == (j - LANES), svals[j], t1)

        loss = jnp.sum(_softplus16(t0)) + jnp.sum(_softplus16(t1))
        out_v[...] = jnp.full((LANES,), loss, jnp.float32)
        pltpu.sync_copy(out_v, out_hbm)


@jax.jit
def _cbow_loss(ctx_idx, wout_idx, w_in, w_out):
    mesh = plsc.VectorSubcoreMesh(core_axis_name="c", subcore_axis_name="s")
    cp = pltpu.CompilerParams()
    if "needs_layout_passes" in pltpu.CompilerParams.__dataclass_fields__:
        cp = dataclasses.replace(cp, needs_layout_passes=False)
    cp = dataclasses.replace(cp, use_tc_tiling_on_sc=False)
    run = pl.kernel(
        _sc_body,
        out_type=jax.ShapeDtypeStruct((LANES,), jnp.float32),
        mesh=mesh,
        scratch_types=[
            pltpu.VMEM((N_CTX,), jnp.int32),
            pltpu.VMEM((N_OUT,), jnp.int32),
            pltpu.VMEM((N_CTX, EMBED), jnp.float32),
            pltpu.VMEM((N_OUT, EMBED), jnp.float32),
            pltpu.VMEM((LANES,), jnp.float32),
            pltpu.SemaphoreType.DMA,
            pltpu.SemaphoreType.DMA,
        ],
        compiler_params=cp,
    )
    return run(ctx_idx, wout_idx, w_in, w_out)[0]


def kernel(context_idxs, target_idx, negative_samples, W_in, W_out):
    ctx_idx = context_idxs.astype(jnp.int32)
    wout_idx = jnp.concatenate(
        [target_idx.reshape(1), negative_samples]).astype(jnp.int32)
    return _cbow_loss(ctx_idx, wout_idx, W_in, W_out)


# speedup vs baseline: 42.7276x; 42.7276x over previous
import dataclasses

import jax
import jax.numpy as jnp
from jax import lax
from jax.experimental import pallas as pl
from jax.experimental.pallas import tpu as pltpu
from jax.experimental.pallas import tpu_sc as plsc

VOCAB = 1000000
EMBED = 64
N_CTX = 50
N_NEG = 20
N_OUT = N_NEG + 1
LANES = 16
N_SUB = 16
CTX_SLOTS = 4
OUT_SLOTS = 2
SLOTS = CTX_SLOTS + OUT_SLOTS
N_PAD = N_SUB * SLOTS
OUT_BASE = N_SUB * CTX_SLOTS
TILE_L = 128


def _softplus16(t):
    m = jnp.maximum(t, 0.0)
    u = jnp.exp(-jnp.abs(t))
    y = 1.0 + u
    x = u * (6.0 + u) / (6.0 + 4.0 * u)
    for _ in range(3):
        x = x + y * jnp.exp(-x) - 1.0
    return m + x


def _sc_body(idx_hbm, wt_in_hbm, wt_out_hbm, out_hbm,
             idx_v, blks_v, col_v, final_v, out_v, stage_shr,
             sem_a, sem_b):
    cid = lax.axis_index("c")
    sub = lax.axis_index("s")

    @pl.when(cid == 0)
    def _():
        io = lax.iota(jnp.int32, LANES)
        pltpu.sync_copy(idx_hbm, idx_v)

        quots = []
        copies = []
        for s in range(SLOTS):
            chunk = idx_v[pl.ds(LANES * s, LANES)]
            r = jnp.sum(jnp.where(io == sub, chunk, 0))
            q128 = pl.multiple_of((r // TILE_L) * TILE_L, TILE_L)
            quots.append(r - q128)
            src = wt_in_hbm if s < CTX_SLOTS else wt_out_hbm
            copies.append(pltpu.async_copy(
                src.at[:, pl.ds(q128, TILE_L)], blks_v.at[s], sem_a))
        for cp_ in copies:
            cp_.wait()

        stages = []
        for s in range(SLOTS):
            colidx = jnp.full((LANES,), quots[s], jnp.int32)
            for c in range(EMBED // LANES):
                vals = plsc.load_gather(
                    blks_v.at[s], [io + LANES * c, colidx])
                col_v[s, pl.ds(LANES * c, LANES)] = vals
            t = sub + LANES * s
            stages.append(
                pltpu.async_copy(col_v.at[s], stage_shr.at[t], sem_b))
        for st in stages:
            st.wait()
        plsc.subcore_barrier()

        @pl.when(sub == 0)
        def _():
            pltpu.sync_copy(stage_shr, final_v)

            h = []
            for c in range(EMBED // LANES):
                acc = final_v[0, pl.ds(LANES * c, LANES)]
                for i in range(1, N_CTX):
                    acc = acc + final_v[i, pl.ds(LANES * c, LANES)]
                h.append(acc * (1.0 / N_CTX))

            svals = []
            for j in range(N_OUT):
                row = OUT_BASE + j
                p = final_v[row, pl.ds(0, LANES)] * h[0]
                for c in range(1, EMBED // LANES):
                    p = p + final_v[row, pl.ds(LANES * c, LANES)] * h[c]
                svals.append(jnp.sum(p))

            t0 = jnp.full((LANES,), -100.0, jnp.float32)
            t1 = jnp.full((LANES,), -100.0, jnp.float32)
            t0 = jnp.where(io == 0, -svals[0], t0)
            for j in range(1, LANES):
                t0 = jnp.where(io == j, svals[j], t0)
            for j in range(LANES, N_OUT):
                t1 = jnp.where(io == (j - LANES), svals[j], t1)

            loss = jnp.sum(_softplus16(t0)) + jnp.sum(_softplus16(t1))
            out_v[...] = jnp.full((LANES,), loss, jnp.float32)
            pltpu.sync_copy(out_v, out_hbm)


@jax.jit
def _cbow_loss(idx, wt_in, wt_out):
    mesh = plsc.VectorSubcoreMesh(core_axis_name="c", subcore_axis_name="s")
    cp = pltpu.CompilerParams()
    if "needs_layout_passes" in pltpu.CompilerParams.__dataclass_fields__:
        cp = dataclasses.replace(cp, needs_layout_passes=False)
    cp = dataclasses.replace(cp, disable_bounds_checks=True)
    run = pl.kernel(
        _sc_body,
        out_type=jax.ShapeDtypeStruct((LANES,), jnp.float32),
        mesh=mesh,
        scratch_types=[
            pltpu.VMEM((N_PAD,), jnp.int32),
            pltpu.VMEM((SLOTS, EMBED, TILE_L), jnp.float32),
            pltpu.VMEM((SLOTS, EMBED), jnp.float32),
            pltpu.VMEM((N_PAD, EMBED), jnp.float32),
            pltpu.VMEM((LANES,), jnp.float32),
            pltpu.VMEM_SHARED((N_PAD, EMBED), jnp.float32),
            pltpu.SemaphoreType.DMA,
            pltpu.SemaphoreType.DMA,
        ],
        compiler_params=cp,
    )
    return run(idx, wt_in, wt_out)[0]


def kernel(context_idxs, target_idx, negative_samples, W_in, W_out):
    idx = jnp.concatenate([
        context_idxs.astype(jnp.int32),
        jnp.zeros((OUT_BASE - N_CTX,), jnp.int32),
        target_idx.reshape(1).astype(jnp.int32),
        negative_samples.astype(jnp.int32),
        jnp.zeros((N_PAD - OUT_BASE - N_OUT,), jnp.int32),
    ])
    return _cbow_loss(idx, W_in.T, W_out.T)
